# Initial kernel scaffold; baseline (speedup 1.0000x reference)
import jax
import jax.numpy as jnp
from jax.experimental import pallas as pl


def kernel(x, edge_index, W1, b1, W2, b2):
    raise NotImplementedError("write your pallas kernel here")


# speedup vs baseline: 1.5338x; 1.5338x over previous
import functools

import jax
import jax.numpy as jnp
from jax import lax
from jax.experimental import pallas as pl
from jax.experimental.pallas import tpu as pltpu
from jax.experimental.pallas import tpu_sc as plsc

N_NODES = 10000
N_EDGES = 160000
D = 256
L = 16
NW = 32
MBLK = 256
N_PAD = 10240
CHUNK = 128
NCHUNK = 40
E_PER_W = CHUNK * NCHUNK
E_PAD = E_PER_W * NW


def _mm_body(x_ref, wa_ref, wb_ref, b1_ref, a_ref, b_ref):
    xb = x_ref[...]
    a_ref[...] = (
        jnp.dot(xb, wa_ref[...], preferred_element_type=jnp.float32)
        + b1_ref[...]
    )
    b_ref[...] = jnp.dot(xb, wb_ref[...], preferred_element_type=jnp.float32)


def _node_tables(x_pad, w1a, w1b, b1row):
    grid = (N_PAD // MBLK,)
    return pl.pallas_call(
        _mm_body,
        grid=grid,
        in_specs=[
            pl.BlockSpec((MBLK, D), lambda i: (i, 0)),
            pl.BlockSpec((D, D), lambda i: (0, 0)),
            pl.BlockSpec((D, D), lambda i: (0, 0)),
            pl.BlockSpec((1, D), lambda i: (0, 0)),
        ],
        out_specs=[
            pl.BlockSpec((MBLK, D), lambda i: (i, 0)),
            pl.BlockSpec((MBLK, D), lambda i: (i, 0)),
        ],
        out_shape=[
            jax.ShapeDtypeStruct((N_PAD, D), jnp.float32),
            jax.ShapeDtypeStruct((N_PAD, D), jnp.float32),
        ],
    )(x_pad, w1a, w1b, b1row)


def _edge_body(a_hbm, b_hbm, src_hbm, dst_hbm, w2b2_hbm, out_hbm,
               srcv, dstv, av, bv, outv, w2v, accbuf, sem):
    wid = lax.axis_index("s") * 2 + lax.axis_index("c")
    base0 = wid * E_PER_W

    pltpu.sync_copy(w2b2_hbm, w2v)
    w2regs = [w2v[pl.ds(j * L, L)] for j in range(D // L)]
    b2vec = w2v[pl.ds(D, L)]
    rows = lax.iota(jnp.int32, L)

    def chunk_body(k, _):
        base = base0 + k * CHUNK
        pltpu.sync_copy(src_hbm.at[pl.ds(base, CHUNK)], srcv)
        pltpu.sync_copy(dst_hbm.at[pl.ds(base, CHUNK)], dstv)
        cp_a = pltpu.async_copy(a_hbm.at[srcv], av, sem)
        cp_b = pltpu.async_copy(b_hbm.at[dstv], bv, sem)
        cp_a.wait()
        cp_b.wait()

        def group_body(g, _):
            def edge_body(i, _):
                e = g * L + i
                acc = b2vec
                for j in range(D // L):
                    h = jnp.maximum(
                        av[e, pl.ds(j * L, L)] + bv[e, pl.ds(j * L, L)], 0.0)
                    acc = acc + h * w2regs[j]
                accbuf[pl.ds(i * L, L)] = acc
                return 0

            lax.fori_loop(0, L, edge_body, 0)
            out16 = plsc.load_gather(accbuf, [rows * L])
            for j in range(1, L):
                out16 = out16 + plsc.load_gather(accbuf, [rows * L + j])
            outv[pl.ds(g * L, L)] = out16
            return 0

        lax.fori_loop(0, CHUNK // L, group_body, 0)
        pltpu.sync_copy(outv, out_hbm.at[pl.ds(base, CHUNK)])
        return 0

    lax.fori_loop(0, NCHUNK, chunk_body, 0)


def _edge_scores(a_tab, b_tab, src_idx, dst_idx, w2b2):
    mesh = plsc.VectorSubcoreMesh(core_axis_name="c", subcore_axis_name="s")
    f = pl.kernel(
        _edge_body,
        out_type=jax.ShapeDtypeStruct((E_PAD,), jnp.float32),
        mesh=mesh,
        compiler_params=pltpu.CompilerParams(needs_layout_passes=False),
        scratch_types=[
            pltpu.VMEM((CHUNK,), jnp.int32),
            pltpu.VMEM((CHUNK,), jnp.int32),
            pltpu.VMEM((CHUNK, D), jnp.float32),
            pltpu.VMEM((CHUNK, D), jnp.float32),
            pltpu.VMEM((CHUNK,), jnp.float32),
            pltpu.VMEM((D + L,), jnp.float32),
            pltpu.VMEM((L * L,), jnp.float32),
            pltpu.SemaphoreType.DMA,
        ],
    )
    return f(a_tab, b_tab, src_idx, dst_idx, w2b2)


@jax.jit
def kernel(x, edge_index, W1, b1, W2, b2):
    x_pad = jnp.pad(x, ((0, N_PAD - N_NODES), (0, 0)))
    w1a = W1[:D, :]
    w1b = W1[D:, :]
    b1row = b1.reshape(1, D)

    a_tab, b_tab = _node_tables(x_pad, w1a, w1b, b1row)

    ei = edge_index.astype(jnp.int32)
    src_idx = jnp.pad(ei[0], (0, E_PAD - N_EDGES))
    dst_idx = jnp.pad(ei[1], (0, E_PAD - N_EDGES))
    w2b2 = jnp.concatenate(
        [W2[:, 0], b2, jnp.zeros((L - 1,), jnp.float32)])

    scores = _edge_scores(a_tab, b_tab, src_idx, dst_idx, w2b2)
    return scores[:N_EDGES]


# speedup vs baseline: 2.3547x; 1.5352x over previous
import functools

import jax
import jax.numpy as jnp
from jax import lax
from jax.experimental import pallas as pl
from jax.experimental.pallas import tpu as pltpu
from jax.experimental.pallas import tpu_sc as plsc

N_NODES = 10000
N_EDGES = 160000
D = 256
L = 16
NW = 32
MBLK = 256
N_PAD = 10240
CHUNK = 64
NCHUNK = 80
E_PER_W = CHUNK * NCHUNK
E_PAD = E_PER_W * NW


def _mm_body(x_ref, wa_ref, wb_ref, b1_ref, a_ref, b_ref):
    xb = x_ref[...]
    a_ref[...] = (
        jnp.dot(xb, wa_ref[...], preferred_element_type=jnp.float32)
        + b1_ref[...]
    )
    b_ref[...] = jnp.dot(xb, wb_ref[...], preferred_element_type=jnp.float32)


def _node_tables(x_pad, w1a, w1b, b1row):
    grid = (N_PAD // MBLK,)
    return pl.pallas_call(
        _mm_body,
        grid=grid,
        in_specs=[
            pl.BlockSpec((MBLK, D), lambda i: (i, 0)),
            pl.BlockSpec((D, D), lambda i: (0, 0)),
            pl.BlockSpec((D, D), lambda i: (0, 0)),
            pl.BlockSpec((1, D), lambda i: (0, 0)),
        ],
        out_specs=[
            pl.BlockSpec((MBLK, D), lambda i: (i, 0)),
            pl.BlockSpec((MBLK, D), lambda i: (i, 0)),
        ],
        out_shape=[
            jax.ShapeDtypeStruct((N_PAD, D), jnp.float32),
            jax.ShapeDtypeStruct((N_PAD, D), jnp.float32),
        ],
    )(x_pad, w1a, w1b, b1row)


def _edge_body(a_hbm, b_hbm, src_hbm, dst_hbm, w2b2_hbm, out_hbm,
               srcall, dstall, av0, av1, bv0, bv1, outv, w2v, accbuf,
               sem0, sem1):
    wid = lax.axis_index("s") * 2 + lax.axis_index("c")
    base0 = wid * E_PER_W
    avs, bvs, sems = (av0, av1), (bv0, bv1), (sem0, sem1)

    pltpu.sync_copy(src_hbm.at[pl.ds(base0, E_PER_W)], srcall)
    pltpu.sync_copy(dst_hbm.at[pl.ds(base0, E_PER_W)], dstall)
    pltpu.sync_copy(w2b2_hbm, w2v)
    w2regs = [w2v[pl.ds(j * L, L)] for j in range(D // L)]
    b2vec = w2v[pl.ds(D, L)]
    rows = lax.iota(jnp.int32, L)

    def start(k, b):
        off = pl.ds(k * CHUNK, CHUNK)
        pltpu.async_copy(a_hbm.at[srcall.at[off]], avs[b], sems[b])
        pltpu.async_copy(b_hbm.at[dstall.at[off]], bvs[b], sems[b])

    def wait(k, b):
        off = pl.ds(k * CHUNK, CHUNK)
        pltpu.make_async_copy(a_hbm.at[srcall.at[off]], avs[b], sems[b]).wait()
        pltpu.make_async_copy(b_hbm.at[dstall.at[off]], bvs[b], sems[b]).wait()

    start(0, 0)

    def outer_body(kk, _):
        for b in range(2):
            k = kk * 2 + b
            av, bv = avs[b], bvs[b]

            @pl.when(k + 1 < NCHUNK)
            def _():
                start(k + 1, 1 - b)

            wait(k, b)

            def group_body(g, _):
                def edge_body(i, _):
                    e = g * L + i
                    acc = b2vec
                    for j in range(D // L):
                        h = jnp.maximum(
                            av[e, pl.ds(j * L, L)] + bv[e, pl.ds(j * L, L)],
                            0.0)
                        acc = acc + h * w2regs[j]
                    accbuf[pl.ds(i * L, L)] = acc
                    return 0

                lax.fori_loop(0, L, edge_body, 0)
                out16 = plsc.load_gather(accbuf, [rows * L])
                for j in range(1, L):
                    out16 = out16 + plsc.load_gather(accbuf, [rows * L + j])
                outv[pl.ds(g * L, L)] = out16
                return 0

            lax.fori_loop(0, CHUNK // L, group_body, 0)
            pltpu.sync_copy(outv, out_hbm.at[pl.ds(base0 + k * CHUNK, CHUNK)])
        return 0

    lax.fori_loop(0, NCHUNK // 2, outer_body, 0)


def _edge_scores(a_tab, b_tab, src_idx, dst_idx, w2b2):
    mesh = plsc.VectorSubcoreMesh(core_axis_name="c", subcore_axis_name="s")
    f = pl.kernel(
        _edge_body,
        out_type=jax.ShapeDtypeStruct((E_PAD,), jnp.float32),
        mesh=mesh,
        compiler_params=pltpu.CompilerParams(needs_layout_passes=False),
        scratch_types=[
            pltpu.VMEM((E_PER_W,), jnp.int32),
            pltpu.VMEM((E_PER_W,), jnp.int32),
            pltpu.VMEM((CHUNK, D), jnp.float32),
            pltpu.VMEM((CHUNK, D), jnp.float32),
            pltpu.VMEM((CHUNK, D), jnp.float32),
            pltpu.VMEM((CHUNK, D), jnp.float32),
            pltpu.VMEM((CHUNK,), jnp.float32),
            pltpu.VMEM((D + L,), jnp.float32),
            pltpu.VMEM((L * L,), jnp.float32),
            pltpu.SemaphoreType.DMA,
            pltpu.SemaphoreType.DMA,
        ],
    )
    return f(a_tab, b_tab, src_idx, dst_idx, w2b2)


@jax.jit
def kernel(x, edge_index, W1, b1, W2, b2):
    x_pad = jnp.pad(x, ((0, N_PAD - N_NODES), (0, 0)))
    w1a = W1[:D, :]
    w1b = W1[D:, :]
    b1row = b1.reshape(1, D)

    a_tab, b_tab = _node_tables(x_pad, w1a, w1b, b1row)

    ei = edge_index.astype(jnp.int32)
    src_idx = jnp.pad(ei[0], (0, E_PAD - N_EDGES))
    dst_idx = jnp.pad(ei[1], (0, E_PAD - N_EDGES))
    w2b2 = jnp.concatenate(
        [W2[:, 0], b2, jnp.zeros((L - 1,), jnp.float32)])

    scores = _edge_scores(a_tab, b_tab, src_idx, dst_idx, w2b2)
    return scores[:N_EDGES]


# speedup vs baseline: 2.7467x; 1.1665x over previous
import functools

import jax
import jax.numpy as jnp
from jax import lax
from jax.experimental import pallas as pl
from jax.experimental.pallas import tpu as pltpu
from jax.experimental.pallas import tpu_sc as plsc

N_NODES = 10000
N_EDGES = 160000
D = 256
L = 16
NS = 16
MBLK = 400
CHUNK = 64
K0 = 104
K1 = 56
KMAX = max(K0, K1)
E_PAD = (K0 + K1) * NS * CHUNK
E_IDX_PAD = (NS * K0 + (NS - 1) * K1 + KMAX) * CHUNK


def _mm_body(x_ref, wa_ref, wb_ref, b1_ref, a_ref, b_ref):
    xb = x_ref[...]
    a_ref[...] = (
        jnp.dot(xb, wa_ref[...], preferred_element_type=jnp.float32)
        + b1_ref[...]
    )
    b_ref[...] = jnp.dot(xb, wb_ref[...], preferred_element_type=jnp.float32)


def _node_tables(x_pad, w1a, w1b, b1row):
    grid = (N_NODES // MBLK,)
    return pl.pallas_call(
        _mm_body,
        grid=grid,
        in_specs=[
            pl.BlockSpec((MBLK, D), lambda i: (i, 0)),
            pl.BlockSpec((D, D), lambda i: (0, 0)),
            pl.BlockSpec((D, D), lambda i: (0, 0)),
            pl.BlockSpec((1, D), lambda i: (0, 0)),
        ],
        out_specs=[
            pl.BlockSpec((MBLK, D), lambda i: (i, 0)),
            pl.BlockSpec((MBLK, D), lambda i: (i, 0)),
        ],
        out_shape=[
            jax.ShapeDtypeStruct((N_NODES, D), jnp.float32),
            jax.ShapeDtypeStruct((N_NODES, D), jnp.float32),
        ],
    )(x_pad, w1a, w1b, b1row)


def _edge_body(a_hbm, b_hbm, src_hbm, dst_hbm, w2b2_hbm, out_hbm,
               srcall, dstall, av0, av1, bv0, bv1, outv, w2v, accbuf,
               sem0, sem1):
    c = lax.axis_index("c")
    s = lax.axis_index("s")
    is0 = c == 0
    nchunk = jnp.where(is0, K0, K1)
    base0 = jnp.where(
        is0, s * (K0 * CHUNK), NS * K0 * CHUNK + s * (K1 * CHUNK))
    avs, bvs, sems = (av0, av1), (bv0, bv1), (sem0, sem1)

    pltpu.sync_copy(src_hbm.at[pl.ds(base0, KMAX * CHUNK)], srcall)
    pltpu.sync_copy(dst_hbm.at[pl.ds(base0, KMAX * CHUNK)], dstall)
    pltpu.sync_copy(w2b2_hbm, w2v)
    w2regs = [w2v[pl.ds(j * L, L)] for j in range(D // L)]
    b2vec = w2v[pl.ds(D, L)]
    rows = lax.iota(jnp.int32, L)

    def start(k, b):
        off = pl.ds(k * CHUNK, CHUNK)
        pltpu.async_copy(a_hbm.at[srcall.at[off]], avs[b], sems[b])
        pltpu.async_copy(b_hbm.at[dstall.at[off]], bvs[b], sems[b])

    def wait(k, b):
        off = pl.ds(k * CHUNK, CHUNK)
        pltpu.make_async_copy(a_hbm.at[srcall.at[off]], avs[b], sems[b]).wait()
        pltpu.make_async_copy(b_hbm.at[dstall.at[off]], bvs[b], sems[b]).wait()

    start(0, 0)

    def outer_body(kk, _):
        for b in range(2):
            k = kk * 2 + b
            av, bv = avs[b], bvs[b]

            @pl.when(k + 1 < nchunk)
            def _():
                start(k + 1, 1 - b)

            wait(k, b)

            def group_body(g, _):
                def edge_body(i, _):
                    e = g * L + i
                    acc = b2vec
                    for j in range(D // L):
                        h = jnp.maximum(
                            av[e, pl.ds(j * L, L)] + bv[e, pl.ds(j * L, L)],
                            0.0)
                        acc = acc + h * w2regs[j]
                    accbuf[pl.ds(i * L, L)] = acc
                    return 0

                lax.fori_loop(0, L, edge_body, 0)
                out16 = plsc.load_gather(accbuf, [rows * L])
                for j in range(1, L):
                    out16 = out16 + plsc.load_gather(accbuf, [rows * L + j])
                outv[pl.ds(g * L, L)] = out16
                return 0

            lax.fori_loop(0, CHUNK // L, group_body, 0)
            pltpu.sync_copy(outv, out_hbm.at[pl.ds(base0 + k * CHUNK, CHUNK)])
        return 0

    lax.fori_loop(0, nchunk // 2, outer_body, 0)


def _edge_scores(a_tab, b_tab, src_idx, dst_idx, w2b2):
    mesh = plsc.VectorSubcoreMesh(core_axis_name="c", subcore_axis_name="s")
    f = pl.kernel(
        _edge_body,
        out_type=jax.ShapeDtypeStruct((E_PAD,), jnp.float32),
        mesh=mesh,
        compiler_params=pltpu.CompilerParams(needs_layout_passes=False),
        scratch_types=[
            pltpu.VMEM((KMAX * CHUNK,), jnp.int32),
            pltpu.VMEM((KMAX * CHUNK,), jnp.int32),
            pltpu.VMEM((CHUNK, D), jnp.float32),
            pltpu.VMEM((CHUNK, D), jnp.float32),
            pltpu.VMEM((CHUNK, D), jnp.float32),
            pltpu.VMEM((CHUNK, D), jnp.float32),
            pltpu.VMEM((CHUNK,), jnp.float32),
            pltpu.VMEM((D + L,), jnp.float32),
            pltpu.VMEM((L * L,), jnp.float32),
            pltpu.SemaphoreType.DMA,
            pltpu.SemaphoreType.DMA,
        ],
    )
    return f(a_tab, b_tab, src_idx, dst_idx, w2b2)


@jax.jit
def kernel(x, edge_index, W1, b1, W2, b2):
    w1a = W1[:D, :]
    w1b = W1[D:, :]
    b1row = b1.reshape(1, D)

    a_tab, b_tab = _node_tables(x, w1a, w1b, b1row)

    ei = edge_index.astype(jnp.int32)
    src_idx = jnp.pad(ei[0], (0, E_IDX_PAD - N_EDGES))
    dst_idx = jnp.pad(ei[1], (0, E_IDX_PAD - N_EDGES))
    w2b2 = jnp.concatenate(
        [W2[:, 0], b2, jnp.zeros((L - 1,), jnp.float32)])

    scores = _edge_scores(a_tab, b_tab, src_idx, dst_idx, w2b2)
    return scores[:N_EDGES]


# speedup vs baseline: 3.0843x; 1.1229x over previous
import functools

import jax
import jax.numpy as jnp
from jax import lax
from jax.experimental import pallas as pl
from jax.experimental.pallas import tpu as pltpu
from jax.experimental.pallas import tpu_sc as plsc

N_NODES = 10000
N_EDGES = 160000
D = 256
L = 16
NS = 16
MBLK = 400
CHUNK = 48
NBUF = 4
K0 = 136
K1 = 76
KMAX = max(K0, K1)
E_PAD = (K0 + K1) * NS * CHUNK
E_IDX_PAD = (NS * K0 + (NS - 1) * K1 + KMAX) * CHUNK


def _mm_body(x_ref, wa_ref, wb_ref, b1_ref, a_ref, b_ref):
    xb = x_ref[...]
    a_ref[...] = (
        jnp.dot(xb, wa_ref[...], preferred_element_type=jnp.float32)
        + b1_ref[...]
    )
    b_ref[...] = jnp.dot(xb, wb_ref[...], preferred_element_type=jnp.float32)


def _node_tables(x_pad, w1a, w1b, b1row):
    grid = (N_NODES // MBLK,)
    return pl.pallas_call(
        _mm_body,
        grid=grid,
        in_specs=[
            pl.BlockSpec((MBLK, D), lambda i: (i, 0)),
            pl.BlockSpec((D, D), lambda i: (0, 0)),
            pl.BlockSpec((D, D), lambda i: (0, 0)),
            pl.BlockSpec((1, D), lambda i: (0, 0)),
        ],
        out_specs=[
            pl.BlockSpec((MBLK, D), lambda i: (i, 0)),
            pl.BlockSpec((MBLK, D), lambda i: (i, 0)),
        ],
        out_shape=[
            jax.ShapeDtypeStruct((N_NODES, D), jnp.float32),
            jax.ShapeDtypeStruct((N_NODES, D), jnp.float32),
        ],
    )(x_pad, w1a, w1b, b1row)


def _edge_body(a_hbm, b_hbm, src_hbm, dst_hbm, w2b2_hbm, out_hbm,
               srcall, dstall, outv, w2v, accbuf, *bufs_and_sems):
    avs = bufs_and_sems[0:NBUF]
    bvs = bufs_and_sems[NBUF:2 * NBUF]
    sems = bufs_and_sems[2 * NBUF:3 * NBUF]
    c = lax.axis_index("c")
    s = lax.axis_index("s")
    is0 = c == 0
    nchunk = jnp.where(is0, K0, K1)
    base0 = jnp.where(
        is0, s * (K0 * CHUNK), NS * K0 * CHUNK + s * (K1 * CHUNK))

    pltpu.sync_copy(src_hbm.at[pl.ds(base0, KMAX * CHUNK)], srcall)
    pltpu.sync_copy(dst_hbm.at[pl.ds(base0, KMAX * CHUNK)], dstall)
    pltpu.sync_copy(w2b2_hbm, w2v)
    w2regs = [w2v[pl.ds(j * L, L)] for j in range(D // L)]
    b2vec = w2v[pl.ds(D, L)]
    rows = lax.iota(jnp.int32, L)

    def start(k, b):
        off = pl.ds(k * CHUNK, CHUNK)
        pltpu.async_copy(a_hbm.at[srcall.at[off]], avs[b], sems[b])
        pltpu.async_copy(b_hbm.at[dstall.at[off]], bvs[b], sems[b])

    def wait(k, b):
        off = pl.ds(k * CHUNK, CHUNK)
        pltpu.make_async_copy(a_hbm.at[srcall.at[off]], avs[b], sems[b]).wait()
        pltpu.make_async_copy(b_hbm.at[dstall.at[off]], bvs[b], sems[b]).wait()

    for b in range(NBUF - 1):
        start(b, b)

    def outer_body(kk, _):
        for b in range(NBUF):
            k = kk * NBUF + b
            av, bv = avs[b], bvs[b]

            @pl.when(k + NBUF - 1 < nchunk)
            def _():
                start(k + NBUF - 1, (b + NBUF - 1) % NBUF)

            wait(k, b)

            def group_body(g, _):
                def edge_body(i, _):
                    e = g * L + i
                    acc = b2vec
                    for j in range(D // L):
                        h = jnp.maximum(
                            av[e, pl.ds(j * L, L)] + bv[e, pl.ds(j * L, L)],
                            0.0)
                        acc = acc + h * w2regs[j]
                    accbuf[pl.ds(i * L, L)] = acc
                    return 0

                lax.fori_loop(0, L, edge_body, 0)
                out16 = plsc.load_gather(accbuf, [rows * L])
                for j in range(1, L):
                    out16 = out16 + plsc.load_gather(accbuf, [rows * L + j])
                outv[pl.ds(g * L, L)] = out16
                return 0

            lax.fori_loop(0, CHUNK // L, group_body, 0)
            pltpu.sync_copy(outv, out_hbm.at[pl.ds(base0 + k * CHUNK, CHUNK)])
        return 0

    lax.fori_loop(0, nchunk // NBUF, outer_body, 0)


def _edge_scores(a_tab, b_tab, src_idx, dst_idx, w2b2):
    mesh = plsc.VectorSubcoreMesh(core_axis_name="c", subcore_axis_name="s")
    f = pl.kernel(
        _edge_body,
        out_type=jax.ShapeDtypeStruct((E_PAD,), jnp.float32),
        mesh=mesh,
        compiler_params=pltpu.CompilerParams(needs_layout_passes=False),
        scratch_types=(
            [
                pltpu.VMEM((KMAX * CHUNK,), jnp.int32),
                pltpu.VMEM((KMAX * CHUNK,), jnp.int32),
                pltpu.VMEM((CHUNK,), jnp.float32),
                pltpu.VMEM((D + L,), jnp.float32),
                pltpu.VMEM((L * L,), jnp.float32),
            ]
            + [pltpu.VMEM((CHUNK, D), jnp.float32)] * (2 * NBUF)
            + [pltpu.SemaphoreType.DMA] * NBUF
        ),
    )
    return f(a_tab, b_tab, src_idx, dst_idx, w2b2)


@jax.jit
def kernel(x, edge_index, W1, b1, W2, b2):
    w1a = W1[:D, :]
    w1b = W1[D:, :]
    b1row = b1.reshape(1, D)

    a_tab, b_tab = _node_tables(x, w1a, w1b, b1row)

    ei = edge_index.astype(jnp.int32)
    src_idx = jnp.pad(ei[0], (0, E_IDX_PAD - N_EDGES))
    dst_idx = jnp.pad(ei[1], (0, E_IDX_PAD - N_EDGES))
    w2b2 = jnp.concatenate(
        [W2[:, 0], b2, jnp.zeros((L - 1,), jnp.float32)])

    scores = _edge_scores(a_tab, b_tab, src_idx, dst_idx, w2b2)
    return scores[:N_EDGES]
